# baseline (speedup 1.0000x reference)
import functools

import numpy as np
import jax
import jax.numpy as jnp
from jax import lax
from jax.experimental import pallas as pl
from jax.experimental.pallas import tpu as pltpu
from jax.experimental.pallas import tpu_sc as plsc

_NS = 2
_B = 64
_H = 100000
_R = _NS * _B
_T = 1024
_TC = _T // 128

_HP = 100352
_TINY = np.float32(np.finfo(np.float32).tiny)
_SCALE = np.float32(np.float32(1.0) - _TINY)
_NEG = np.float32(-np.inf)
_BIG = np.int32(2147483647)


def _rotl(x, r):
    return (x << np.uint32(r)) | (x >> np.uint32(32 - r))


def _threefry_rounds(x0, x1, rots):
    for r in rots:
        x0 = x0 + x1
        x1 = _rotl(x1, r)
        x1 = x0 ^ x1
    return x0, x1


def _gumbel_from_flat(flat):
    k1 = np.uint32(0)
    k2 = np.uint32(42)
    ks = [k1, k2, np.uint32(k1 ^ k2 ^ np.uint32(0x1BD11BDA))]
    rot_a = (13, 15, 26, 6)
    rot_b = (17, 29, 16, 24)

    x0 = jnp.zeros_like(flat) + ks[0]
    x1 = flat + ks[1]
    x0, x1 = _threefry_rounds(x0, x1, rot_a)
    x0 = x0 + ks[1]
    x1 = x1 + (ks[2] + np.uint32(1))
    x0, x1 = _threefry_rounds(x0, x1, rot_b)
    x0 = x0 + ks[2]
    x1 = x1 + (ks[0] + np.uint32(2))
    x0, x1 = _threefry_rounds(x0, x1, rot_a)
    x0 = x0 + ks[0]
    x1 = x1 + (ks[1] + np.uint32(3))
    x0, x1 = _threefry_rounds(x0, x1, rot_b)
    x0 = x0 + ks[1]
    x1 = x1 + (ks[2] + np.uint32(4))
    x0, x1 = _threefry_rounds(x0, x1, rot_a)
    x0 = x0 + ks[2]
    x1 = x1 + (ks[0] + np.uint32(5))
    bits = x0 ^ x1

    float_bits = (bits >> np.uint32(9)) | np.uint32(0x3F800000)
    f = jax.lax.bitcast_convert_type(float_bits, jnp.float32) - jnp.float32(1.0)
    u = jnp.maximum(_TINY, f * _SCALE + _TINY)
    return -jnp.log(-jnp.log(u))


def _np_uniform():
    i = np.arange(_R * _H, dtype=np.uint32)
    k1, k2 = np.uint32(0), np.uint32(42)
    ks = [k1, k2, np.uint32(k1 ^ k2 ^ np.uint32(0x1BD11BDA))]

    def rounds(x0, x1, rots):
        for r in rots:
            x0 = (x0 + x1).astype(np.uint32)
            x1 = ((x1 << np.uint32(r)) | (x1 >> np.uint32(32 - r))).astype(
                np.uint32
            )
            x1 = x0 ^ x1
        return x0, x1

    x0 = np.zeros_like(i) + ks[0]
    x1 = (i + ks[1]).astype(np.uint32)
    inj = [(ks[1], ks[2], 1), (ks[2], ks[0], 2), (ks[0], ks[1], 3),
           (ks[1], ks[2], 4), (ks[2], ks[0], 5)]
    rots = [(13, 15, 26, 6), (17, 29, 16, 24)]
    for n, (a, bb, c) in enumerate(inj):
        x0, x1 = rounds(x0, x1, rots[n % 2])
        x0 = (x0 + a).astype(np.uint32)
        x1 = (x1 + bb + np.uint32(c)).astype(np.uint32)
    bits = x0 ^ x1
    fb = ((bits >> np.uint32(9)) | np.uint32(0x3F800000)).astype(np.uint32)
    f = fb.view(np.float32) - np.float32(1.0)
    u = np.maximum(_TINY, (f * _SCALE + _TINY).astype(np.float32))
    return u.reshape(_R, _H)


_cand_cache = []


def _candidates():
    if not _cand_cache:
        u = _np_uniform()
        part = np.argpartition(u, _H - _T, axis=1)[:, _H - _T:]
        pu = np.take_along_axis(u, part, axis=1)
        order = np.argsort(-pu, axis=1, kind="stable")
        idxs = np.take_along_axis(part, order, axis=1)
        cu = np.take_along_axis(pu, order, axis=1)
        b = (np.arange(_R, dtype=np.int64) % _B)[:, None]
        v = idxs
        pos = (((b // 8) * 13 + v // 8192) * 65536
               + ((v % 8192) // 128) * 1024 + (b % 8) * 128 + (v % 128))
        _cand_cache.append(
            (pos.astype(np.int32).reshape(_R, _TC, 128),
             v.astype(np.int32).reshape(_R, _TC, 128),
             cu.reshape(_R, _TC, 128))
        )
    return _cand_cache[0]


_VBLK = 8192
_NBLK = pl.cdiv(_H, _VBLK)


def _zprep_body(z_ref, cu_ref, zlin_ref, zmax_ref, cg_ref):
    i = pl.program_id(0)
    k = pl.program_id(1)
    blk = z_ref[...]
    zlin_ref[...] = blk.reshape(8 * _VBLK)
    pos = jax.lax.broadcasted_iota(jnp.int32, (8, _VBLK), 1) + k * _VBLK
    m = jnp.max(jnp.where(pos < _H, blk, _NEG), axis=1)
    mb = jnp.broadcast_to(m[:, None], (8, 128))

    @pl.when(k == 0)
    def _():
        zmax_ref[pl.ds(i * 8, 8), :] = mb

    @pl.when(k > 0)
    def _():
        zmax_ref[pl.ds(i * 8, 8), :] = jnp.maximum(
            zmax_ref[pl.ds(i * 8, 8), :], mb
        )

    @pl.when((i == 0) & (k == 0))
    def _():
        cg_ref[...] = -jnp.log(-jnp.log(cu_ref[...]))


def _zprep(z, cu):
    return pl.pallas_call(
        _zprep_body,
        grid=(_B // 8, _NBLK),
        in_specs=[
            pl.BlockSpec((8, _VBLK), lambda i, k: (i, k)),
            pl.BlockSpec((_R, _TC, 128), lambda i, k: (0, 0, 0)),
        ],
        out_specs=[
            pl.BlockSpec((8 * _VBLK,), lambda i, k: (i * _NBLK + k,)),
            pl.BlockSpec((_B, 128), lambda i, k: (0, 0)),
            pl.BlockSpec((_R, _TC, 128), lambda i, k: (0, 0, 0)),
        ],
        out_shape=[
            jax.ShapeDtypeStruct(((_B // 8) * _NBLK * 8 * _VBLK,), jnp.float32),
            jax.ShapeDtypeStruct((_B, 128), jnp.float32),
            jax.ShapeDtypeStruct((_R, _TC, 128), jnp.float32),
        ],
    )(z, cu)


def _lane_shuffle(x, perm):
    dnums = jax.lax.GatherDimensionNumbers(
        offset_dims=(), collapsed_slice_dims=(0,), start_index_map=(0,)
    )
    return jax.lax.gather(
        x, perm[:, None], dnums, (1,),
        mode=jax.lax.GatherScatterMode.PROMISE_IN_BOUNDS,
    )


def _lane_allreduce(x, op):
    lanes = jax.lax.iota(jnp.int32, 16)
    for sh in (8, 4, 2, 1):
        x = op(x, _lane_shuffle(x, lanes ^ sh))
    return x


def _sc_cand_body(z_hbm, cflat_hbm, cv_hbm, cg_hbm, idx_hbm, l_hbm,
                  idx_scr, v_scr, g_scr, zg_scr, oi_scr, of_scr, sem):
    cw = lax.axis_index("c")
    sw = lax.axis_index("s")
    w = sw * 2 + cw

    pltpu.sync_copy(cflat_hbm.at[pl.ds(4 * w, 4)], idx_scr)
    pltpu.sync_copy(cv_hbm.at[pl.ds(4 * w, 4)], v_scr)
    pltpu.sync_copy(cg_hbm.at[pl.ds(4 * w, 4)], g_scr)

    copies = []
    for j in range(4):
        for c in range(_TC):
            copies.append(
                pltpu.async_copy(
                    z_hbm.at[idx_scr.at[j, c]], zg_scr.at[j, c], sem
                )
            )
    for cp in copies:
        cp.wait()

    lanes = jax.lax.iota(jnp.int32, 16)
    acc_i = jnp.zeros((16,), jnp.int32)
    acc_f = jnp.zeros((16,), jnp.float32)
    for j in range(4):
        best = jnp.full((16,), _NEG, jnp.float32)
        bidx = jnp.full((16,), _BIG, jnp.int32)
        for c in range(_TC):
            for t in range(8):
                zg = zg_scr[j, c, pl.ds(t * 16, 16)]
                gg = g_scr[j, c, pl.ds(t * 16, 16)]
                vv = v_scr[j, c, pl.ds(t * 16, 16)]
                val = zg + gg
                gt = val > best
                eq = (val == best) & (vv < bidx)
                bidx = jnp.where(gt | eq, vv, bidx)
                best = jnp.where(gt, val, best)
        m = _lane_allreduce(best, jnp.maximum)
        rowidx = _lane_allreduce(jnp.where(best == m, bidx, _BIG), jnp.minimum)
        acc_i = jnp.where(lanes == j, rowidx, acc_i)
        acc_f = jnp.where(lanes == j, m, acc_f)

    oi_scr[...] = acc_i
    of_scr[...] = acc_f
    pltpu.sync_copy(oi_scr, idx_hbm.at[w])
    pltpu.sync_copy(of_scr, l_hbm.at[w])


_sc_kernel_cache = []


def _sc_candidates(zlin, cflat, cv, cg):
    if not _sc_kernel_cache:
        _sc_kernel_cache.append(
            pl.kernel(
                _sc_cand_body,
                out_type=(
                    jax.ShapeDtypeStruct((32, 16), jnp.int32),
                    jax.ShapeDtypeStruct((32, 16), jnp.float32),
                ),
                mesh=plsc.VectorSubcoreMesh(
                    core_axis_name="c", subcore_axis_name="s",
                    num_cores=2, num_subcores=16,
                ),
                scratch_types=[
                    pltpu.VMEM((4, _TC, 128), jnp.int32),
                    pltpu.VMEM((4, _TC, 128), jnp.int32),
                    pltpu.VMEM((4, _TC, 128), jnp.float32),
                    pltpu.VMEM((4, _TC, 128), jnp.float32),
                    pltpu.VMEM((16,), jnp.int32),
                    pltpu.VMEM((16,), jnp.float32),
                    pltpu.SemaphoreType.DMA,
                ],
            )
        )
    return _sc_kernel_cache[0](zlin, cflat, cv, cg)


def _full_body(z_ref, idx_ref, bv_ref, bi_ref):
    k = pl.program_id(0)
    pos = jax.lax.broadcasted_iota(jnp.int32, (_B, _VBLK), 1) + k * _VBLK
    valid = pos < _H
    b_i = jax.lax.broadcasted_iota(jnp.int32, (_B, _VBLK), 0)
    zb = z_ref[...]
    for s in range(_NS):
        flat = ((s * _B + b_i) * _H + pos).astype(jnp.uint32)
        val = jnp.where(valid, zb + _gumbel_from_flat(flat), _NEG)
        m = jnp.max(val, axis=1)
        cand = jnp.min(jnp.where(val == m[:, None], pos, _BIG), axis=1)

        @pl.when(k == 0)
        def _init(s=s, m=m, cand=cand):
            bv_ref[s, :] = m
            bi_ref[s, :] = cand

        @pl.when(k > 0)
        def _merge(s=s, m=m, cand=cand):
            better = m > bv_ref[s, :]
            bv_ref[s, :] = jnp.where(better, m, bv_ref[s, :])
            bi_ref[s, :] = jnp.where(better, cand, bi_ref[s, :])

    @pl.when(k == _NBLK - 1)
    def _emit():
        idx_ref[...] = bi_ref[...]


def _full_argmax(z):
    return pl.pallas_call(
        _full_body,
        grid=(_NBLK,),
        in_specs=[pl.BlockSpec((_B, _VBLK), lambda k: (0, k))],
        out_specs=pl.BlockSpec((_NS, _B), lambda k: (0, 0)),
        out_shape=jax.ShapeDtypeStruct((_NS, _B), jnp.int32),
        scratch_shapes=[
            pltpu.VMEM((_NS, _B), jnp.float32),
            pltpu.VMEM((_NS, _B), jnp.int32),
        ],
    )(z)


def _onehot_body(idx_ref, out_ref):
    s = pl.program_id(0)
    k = pl.program_id(1)
    pos = jax.lax.broadcasted_iota(jnp.int32, (_B, _VBLK), 1) + k * _VBLK
    sel = idx_ref[s, :][:, None] == pos
    out_ref[...] = sel.astype(jnp.float32)[None]


def _onehot(idx):
    return pl.pallas_call(
        _onehot_body,
        grid=(_NS, _NBLK),
        in_specs=[pl.BlockSpec((_NS, _B), lambda s, k: (0, 0))],
        out_specs=pl.BlockSpec((1, _B, _VBLK), lambda s, k: (s, 0, k)),
        out_shape=jax.ShapeDtypeStruct((_NS, _B, _H), jnp.float32),
    )(idx)


def kernel(z):
    cpos_np, cv_np, cu_np = _candidates()
    zlin, zmax2, cg = _zprep(z, cu_np)
    zmax = zmax2[:, 0]
    idx_o, l_o = _sc_candidates(zlin, cpos_np, cv_np, cg)
    idx_sb = idx_o[:, :4].reshape(_NS, _B)
    l_sb = l_o[:, :4].reshape(_NS, _B)
    gT = cg[:, _TC - 1, 127].reshape(_NS, _B)
    ok = jnp.all(l_sb > zmax[None, :] + gT)
    idx_final = jax.lax.cond(ok, lambda zz: idx_sb, _full_argmax, z)
    return _onehot(idx_final)
    idx_o, l_o = _sc_candidates(z.reshape(-1), cflat_np, cg)
    idx_sb = idx_o[:, :4].reshape(_NS, _B)
    l_sb = l_o[:, :4].reshape(_NS, _B)
    gT = cg[:, _TC - 1, 127].reshape(_NS, _B)
    ok = jnp.all(l_sb > zmax[None, :] + gT)
    idx_final = jax.lax.cond(ok, lambda zz: idx_sb, _full_argmax, z)
    return _onehot(idx_final)


# speedup vs baseline: 1.2198x; 1.2198x over previous
import functools

import numpy as np
import jax
import jax.numpy as jnp
from jax import lax
from jax.experimental import pallas as pl
from jax.experimental.pallas import tpu as pltpu
from jax.experimental.pallas import tpu_sc as plsc

_NS = 2
_B = 64
_H = 100000
_R = _NS * _B
_T = 1024
_TC = _T // 128

_TINY = np.float32(np.finfo(np.float32).tiny)
_SCALE = np.float32(np.float32(1.0) - _TINY)
_NEG = np.float32(-np.inf)
_BIG = np.int32(2147483647)


def _rotl(x, r):
    return (x << np.uint32(r)) | (x >> np.uint32(32 - r))


def _threefry_rounds(x0, x1, rots):
    for r in rots:
        x0 = x0 + x1
        x1 = _rotl(x1, r)
        x1 = x0 ^ x1
    return x0, x1


def _gumbel_from_flat(flat):
    k1 = np.uint32(0)
    k2 = np.uint32(42)
    ks = [k1, k2, np.uint32(k1 ^ k2 ^ np.uint32(0x1BD11BDA))]
    rot_a = (13, 15, 26, 6)
    rot_b = (17, 29, 16, 24)

    x0 = jnp.zeros_like(flat) + ks[0]
    x1 = flat + ks[1]
    x0, x1 = _threefry_rounds(x0, x1, rot_a)
    x0 = x0 + ks[1]
    x1 = x1 + (ks[2] + np.uint32(1))
    x0, x1 = _threefry_rounds(x0, x1, rot_b)
    x0 = x0 + ks[2]
    x1 = x1 + (ks[0] + np.uint32(2))
    x0, x1 = _threefry_rounds(x0, x1, rot_a)
    x0 = x0 + ks[0]
    x1 = x1 + (ks[1] + np.uint32(3))
    x0, x1 = _threefry_rounds(x0, x1, rot_b)
    x0 = x0 + ks[1]
    x1 = x1 + (ks[2] + np.uint32(4))
    x0, x1 = _threefry_rounds(x0, x1, rot_a)
    x0 = x0 + ks[2]
    x1 = x1 + (ks[0] + np.uint32(5))
    bits = x0 ^ x1

    float_bits = (bits >> np.uint32(9)) | np.uint32(0x3F800000)
    f = jax.lax.bitcast_convert_type(float_bits, jnp.float32) - jnp.float32(1.0)
    u = jnp.maximum(_TINY, f * _SCALE + _TINY)
    return -jnp.log(-jnp.log(u))


def _np_uniform():
    i = np.arange(_R * _H, dtype=np.uint32)
    k1, k2 = np.uint32(0), np.uint32(42)
    ks = [k1, k2, np.uint32(k1 ^ k2 ^ np.uint32(0x1BD11BDA))]

    def rounds(x0, x1, rots):
        for r in rots:
            x0 = (x0 + x1).astype(np.uint32)
            x1 = ((x1 << np.uint32(r)) | (x1 >> np.uint32(32 - r))).astype(
                np.uint32
            )
            x1 = x0 ^ x1
        return x0, x1

    x0 = np.zeros_like(i) + ks[0]
    x1 = (i + ks[1]).astype(np.uint32)
    inj = [(ks[1], ks[2], 1), (ks[2], ks[0], 2), (ks[0], ks[1], 3),
           (ks[1], ks[2], 4), (ks[2], ks[0], 5)]
    rots = [(13, 15, 26, 6), (17, 29, 16, 24)]
    for n, (a, bb, c) in enumerate(inj):
        x0, x1 = rounds(x0, x1, rots[n % 2])
        x0 = (x0 + a).astype(np.uint32)
        x1 = (x1 + bb + np.uint32(c)).astype(np.uint32)
    bits = x0 ^ x1
    fb = ((bits >> np.uint32(9)) | np.uint32(0x3F800000)).astype(np.uint32)
    f = fb.view(np.float32) - np.float32(1.0)
    u = np.maximum(_TINY, (f * _SCALE + _TINY).astype(np.float32))
    return u.reshape(_R, _H)


_cand_cache = []


def _candidates():
    if not _cand_cache:
        u = _np_uniform()
        part = np.argpartition(u, _H - _T, axis=1)[:, _H - _T:]
        pu = np.take_along_axis(u, part, axis=1)
        order = np.argsort(-pu, axis=1, kind="stable")
        idxs = np.take_along_axis(part, order, axis=1)
        cu = np.take_along_axis(pu, order, axis=1)
        b = (np.arange(_R, dtype=np.int64) % _B)
        flat = (idxs + (b * _H)[:, None]).astype(np.int32)
        _cand_cache.append(
            (flat.reshape(_R, _TC, 128), cu.reshape(_R, _TC, 128))
        )
    return _cand_cache[0]


_VBLK = 8192
_NBLK = pl.cdiv(_H, _VBLK)


def _zmax_body(z_ref, cu_ref, zmax_ref, cg_ref):
    k = pl.program_id(0)
    pos = jax.lax.broadcasted_iota(jnp.int32, (_B, _VBLK), 1) + k * _VBLK
    m = jnp.max(jnp.where(pos < _H, z_ref[...], _NEG), axis=1)

    @pl.when(k == 0)
    def _():
        zmax_ref[...] = m
        cg_ref[...] = -jnp.log(-jnp.log(cu_ref[...]))

    @pl.when(k > 0)
    def _():
        zmax_ref[...] = jnp.maximum(zmax_ref[...], m)


def _zmax_candg(z, cu):
    return pl.pallas_call(
        _zmax_body,
        grid=(_NBLK,),
        in_specs=[
            pl.BlockSpec((_B, _VBLK), lambda k: (0, k)),
            pl.BlockSpec((_R, _TC, 128), lambda k: (0, 0, 0)),
        ],
        out_specs=[
            pl.BlockSpec((_B,), lambda k: (0,)),
            pl.BlockSpec((_R, _TC, 128), lambda k: (0, 0, 0)),
        ],
        out_shape=[
            jax.ShapeDtypeStruct((_B,), jnp.float32),
            jax.ShapeDtypeStruct((_R, _TC, 128), jnp.float32),
        ],
    )(z, cu)


def _lane_shuffle(x, perm):
    dnums = jax.lax.GatherDimensionNumbers(
        offset_dims=(), collapsed_slice_dims=(0,), start_index_map=(0,)
    )
    return jax.lax.gather(
        x, perm[:, None], dnums, (1,),
        mode=jax.lax.GatherScatterMode.PROMISE_IN_BOUNDS,
    )


def _lane_allreduce(x, op):
    lanes = jax.lax.iota(jnp.int32, 16)
    for sh in (8, 4, 2, 1):
        x = op(x, _lane_shuffle(x, lanes ^ sh))
    return x


def _sc_cand_body(z_hbm, cflat_hbm, cg_hbm, idx_hbm, l_hbm,
                  idx_scr, g_scr, zg_scr, oi_scr, of_scr, sem):
    cw = lax.axis_index("c")
    sw = lax.axis_index("s")
    w = sw * 2 + cw

    pltpu.sync_copy(cflat_hbm.at[pl.ds(4 * w, 4)], idx_scr)
    pltpu.sync_copy(cg_hbm.at[pl.ds(4 * w, 4)], g_scr)

    copies = []
    for j in range(4):
        for c in range(_TC):
            copies.append(
                pltpu.async_copy(z_hbm.at[idx_scr.at[j, c]], zg_scr.at[j, c], sem)
            )
    for cp in copies:
        cp.wait()

    lanes = jax.lax.iota(jnp.int32, 16)
    acc_i = jnp.zeros((16,), jnp.int32)
    acc_f = jnp.zeros((16,), jnp.float32)
    for j in range(4):
        r = 4 * w + j
        b_off = lax.rem(r, _B) * _H
        best = jnp.full((16,), _NEG, jnp.float32)
        bidx = jnp.full((16,), _BIG, jnp.int32)
        for c in range(_TC):
            for t in range(8):
                zg = zg_scr[j, c, pl.ds(t * 16, 16)]
                gg = g_scr[j, c, pl.ds(t * 16, 16)]
                vv = idx_scr[j, c, pl.ds(t * 16, 16)] - b_off
                val = zg + gg
                gt = val > best
                eq = (val == best) & (vv < bidx)
                bidx = jnp.where(gt | eq, vv, bidx)
                best = jnp.where(gt, val, best)
        m = _lane_allreduce(best, jnp.maximum)
        rowidx = _lane_allreduce(jnp.where(best == m, bidx, _BIG), jnp.minimum)
        acc_i = jnp.where(lanes == j, rowidx, acc_i)
        acc_f = jnp.where(lanes == j, m, acc_f)

    oi_scr[...] = acc_i
    of_scr[...] = acc_f
    pltpu.sync_copy(oi_scr, idx_hbm.at[w])
    pltpu.sync_copy(of_scr, l_hbm.at[w])


_sc_kernel_cache = []


def _sc_candidates(z_flat, cflat, cg):
    if not _sc_kernel_cache:
        _sc_kernel_cache.append(
            pl.kernel(
                _sc_cand_body,
                out_type=(
                    jax.ShapeDtypeStruct((32, 16), jnp.int32),
                    jax.ShapeDtypeStruct((32, 16), jnp.float32),
                ),
                mesh=plsc.VectorSubcoreMesh(
                    core_axis_name="c", subcore_axis_name="s",
                    num_cores=2, num_subcores=16,
                ),
                scratch_types=[
                    pltpu.VMEM((4, _TC, 128), jnp.int32),
                    pltpu.VMEM((4, _TC, 128), jnp.float32),
                    pltpu.VMEM((4, _TC, 128), jnp.float32),
                    pltpu.VMEM((16,), jnp.int32),
                    pltpu.VMEM((16,), jnp.float32),
                    pltpu.SemaphoreType.DMA,
                ],
            )
        )
    return _sc_kernel_cache[0](z_flat, cflat, cg)


def _full_body(z_ref, idx_ref, bv_ref, bi_ref):
    k = pl.program_id(0)
    pos = jax.lax.broadcasted_iota(jnp.int32, (_B, _VBLK), 1) + k * _VBLK
    valid = pos < _H
    b_i = jax.lax.broadcasted_iota(jnp.int32, (_B, _VBLK), 0)
    zb = z_ref[...]
    for s in range(_NS):
        flat = ((s * _B + b_i) * _H + pos).astype(jnp.uint32)
        val = jnp.where(valid, zb + _gumbel_from_flat(flat), _NEG)
        m = jnp.max(val, axis=1)
        cand = jnp.min(jnp.where(val == m[:, None], pos, _BIG), axis=1)

        @pl.when(k == 0)
        def _init(s=s, m=m, cand=cand):
            bv_ref[s, :] = m
            bi_ref[s, :] = cand

        @pl.when(k > 0)
        def _merge(s=s, m=m, cand=cand):
            better = m > bv_ref[s, :]
            bv_ref[s, :] = jnp.where(better, m, bv_ref[s, :])
            bi_ref[s, :] = jnp.where(better, cand, bi_ref[s, :])

    @pl.when(k == _NBLK - 1)
    def _emit():
        idx_ref[...] = bi_ref[...]


def _full_argmax(z):
    return pl.pallas_call(
        _full_body,
        grid=(_NBLK,),
        in_specs=[pl.BlockSpec((_B, _VBLK), lambda k: (0, k))],
        out_specs=pl.BlockSpec((_NS, _B), lambda k: (0, 0)),
        out_shape=jax.ShapeDtypeStruct((_NS, _B), jnp.int32),
        scratch_shapes=[
            pltpu.VMEM((_NS, _B), jnp.float32),
            pltpu.VMEM((_NS, _B), jnp.int32),
        ],
    )(z)


def _onehot_body(idx_ref, out_ref):
    s = pl.program_id(0)
    k = pl.program_id(1)
    pos = jax.lax.broadcasted_iota(jnp.int32, (_B, _VBLK), 1) + k * _VBLK
    sel = idx_ref[s, :][:, None] == pos
    out_ref[...] = sel.astype(jnp.float32)[None]


def _onehot(idx):
    return pl.pallas_call(
        _onehot_body,
        grid=(_NS, _NBLK),
        in_specs=[pl.BlockSpec((_NS, _B), lambda s, k: (0, 0))],
        out_specs=pl.BlockSpec((1, _B, _VBLK), lambda s, k: (s, 0, k)),
        out_shape=jax.ShapeDtypeStruct((_NS, _B, _H), jnp.float32),
    )(idx)


def kernel(z):
    cflat_np, cu_np = _candidates()
    zmax, cg = _zmax_candg(z, cu_np)
    idx_o, l_o = _sc_candidates(z.reshape(-1), cflat_np, cg)
    idx_sb = idx_o[:, :4].reshape(_NS, _B)
    l_sb = l_o[:, :4].reshape(_NS, _B)
    gT = cg[:, _TC - 1, 127].reshape(_NS, _B)
    ok = jnp.all(l_sb > zmax[None, :] + gT)
    idx_final = jax.lax.cond(ok, lambda zz: idx_sb, _full_argmax, z)
    return _onehot(idx_final)
